# Initial kernel scaffold; baseline (speedup 1.0000x reference)
import jax
import jax.numpy as jnp
from jax.experimental import pallas as pl


def kernel(frames, enc_w1, enc_b1, enc_w2, enc_b2, enc_w3, enc_b3, codebook, dec_w1, dec_b1, dec_w2, dec_b2, dec_w3, dec_b3):
    raise NotImplementedError("write your pallas kernel here")


# speedup vs baseline: 2.0948x; 2.0948x over previous
import functools

import jax
import jax.numpy as jnp
from jax.experimental import pallas as pl

B, H, W, C = 8, 224, 224, 3
PS = 4
VOCAB = 512
D = 256
PD = PS * PS * C
Hp = H // PS
Wp = W // PS
N = Hp * Wp
R = B * N

BLK = 1568
G = R // BLK


_INV_SQRT2 = 0.7071067811865476


def _gelu(x):
    return x * 0.5 * (1.0 + jax.lax.erf(x * _INV_SQRT2))


def _fused_body(praw_ref, ew1, eb1, ew2, eb2, ew3, eb3, cb, dw1, db1, dw2,
                db2, dw3, db3, p_out, tok_out, loss_out):
    i = pl.program_id(0)
    t = praw_ref[...] / 255.0 * 2.0 - 1.0
    z = _gelu(jnp.dot(t, ew1[...], preferred_element_type=jnp.float32) + eb1[...])
    z = _gelu(jnp.dot(z, ew2[...], preferred_element_type=jnp.float32) + eb2[...])
    z_e = jnp.dot(z, ew3[...], preferred_element_type=jnp.float32) + eb3[...]

    codebook = cb[...]
    zn = jnp.sum(z_e * z_e, axis=-1, keepdims=True)
    cn = jnp.sum(codebook * codebook, axis=-1)
    score = jnp.dot(z_e, codebook.T, preferred_element_type=jnp.float32)
    d2 = zn + cn[None, :] - 2.0 * score

    m = jnp.min(d2, axis=-1, keepdims=True)
    iota = jax.lax.broadcasted_iota(jnp.int32, d2.shape, 1)
    tok = jnp.min(jnp.where(d2 == m, iota, VOCAB), axis=-1)
    tok_out[0, 0, :] = tok

    onehot = (iota == tok[:, None]).astype(jnp.float32)
    z_q = jnp.dot(onehot, codebook, preferred_element_type=jnp.float32)
    zq_st = z_e + (z_q - z_e)

    x = _gelu(jnp.dot(zq_st, dw1[...], preferred_element_type=jnp.float32) + db1[...])
    x = _gelu(jnp.dot(x, dw2[...], preferred_element_type=jnp.float32) + db2[...])
    p = jnp.dot(x, dw3[...], preferred_element_type=jnp.float32) + db3[...]
    p_out[...] = p

    vq_sum = jnp.sum((z_e - z_q) ** 2)
    rec_sum = jnp.sum((p - t) ** 2)

    @pl.when(i == 0)
    def _init():
        loss_out[...] = jnp.zeros_like(loss_out)

    upd = jnp.concatenate([rec_sum.reshape(1, 1), vq_sum.reshape(1, 1)], axis=1)
    loss_out[...] += upd


@functools.partial(jax.jit, static_argnames=())
def kernel(frames, enc_w1, enc_b1, enc_w2, enc_b2, enc_w3, enc_b3, codebook,
           dec_w1, dec_b1, dec_w2, dec_b2, dec_w3, dec_b3):
    praw = frames.astype(jnp.float32).reshape(B, Hp, PS, Wp, PS, C)
    praw = praw.transpose(0, 1, 3, 2, 4, 5).reshape(R, PD)

    bspecs = [
        pl.BlockSpec((BLK, PD), lambda i: (i, 0)),
        pl.BlockSpec((PD, D), lambda i: (0, 0)),
        pl.BlockSpec((1, D), lambda i: (0, 0)),
        pl.BlockSpec((D, D), lambda i: (0, 0)),
        pl.BlockSpec((1, D), lambda i: (0, 0)),
        pl.BlockSpec((D, D), lambda i: (0, 0)),
        pl.BlockSpec((1, D), lambda i: (0, 0)),
        pl.BlockSpec((VOCAB, D), lambda i: (0, 0)),
        pl.BlockSpec((D, D), lambda i: (0, 0)),
        pl.BlockSpec((1, D), lambda i: (0, 0)),
        pl.BlockSpec((D, D), lambda i: (0, 0)),
        pl.BlockSpec((1, D), lambda i: (0, 0)),
        pl.BlockSpec((D, PD), lambda i: (0, 0)),
        pl.BlockSpec((1, PD), lambda i: (0, 0)),
    ]
    out_shapes = (
        jax.ShapeDtypeStruct((R, PD), jnp.float32),
        jax.ShapeDtypeStruct((G, 1, BLK), jnp.int32),
        jax.ShapeDtypeStruct((1, 2), jnp.float32),
    )
    out_specs = (
        pl.BlockSpec((BLK, PD), lambda i: (i, 0)),
        pl.BlockSpec((1, 1, BLK), lambda i: (i, 0, 0)),
        pl.BlockSpec((1, 2), lambda i: (0, 0)),
    )
    p_full, tok3, sums = pl.pallas_call(
        _fused_body,
        grid=(G,),
        in_specs=bspecs,
        out_specs=out_specs,
        out_shape=out_shapes,
    )(praw, enc_w1, enc_b1.reshape(1, D), enc_w2, enc_b2.reshape(1, D),
      enc_w3, enc_b3.reshape(1, D), codebook, dec_w1, dec_b1.reshape(1, D),
      dec_w2, dec_b2.reshape(1, D), dec_w3, dec_b3.reshape(1, PD))

    tokens = tok3.reshape(B, N)
    recon = p_full.reshape(B, Hp, Wp, PS, PS, C).transpose(0, 1, 3, 2, 4, 5)
    recon = recon.reshape(B, H, W, C)
    recon_loss = sums[0, 0] / (B * H * W * C)
    vq_loss = sums[0, 1] / (R * D)
    return (recon, tokens, recon_loss, vq_loss, vq_loss)


# speedup vs baseline: 2.1629x; 1.0325x over previous
import jax
import jax.numpy as jnp
from jax.experimental import pallas as pl

B, H, W, C = 8, 224, 224, 3
PS = 4
VOCAB = 512
D = 256
PD = PS * PS * C
Hp = H // PS
Wp = W // PS
N = Hp * Wp
R = B * N

BLK = 1568
G = R // BLK

_INV_SQRT2 = 0.7071067811865476


def _gelu(x):
    return x * 0.5 * (1.0 + jax.lax.erf(x * _INV_SQRT2))


def _table_body(cb, dw1, db1, dw2, db2, dw3, db3, ptable_out, cn_out):
    codebook = cb[...]
    cn_out[...] = jnp.sum(codebook * codebook, axis=-1)[None, :]
    x = _gelu(jnp.dot(codebook, dw1[...], preferred_element_type=jnp.float32) + db1[...])
    x = _gelu(jnp.dot(x, dw2[...], preferred_element_type=jnp.float32) + db2[...])
    ptable_out[...] = jnp.dot(x, dw3[...], preferred_element_type=jnp.float32) + db3[...]


def _main_body(praw_ref, ew1, eb1, ew2, eb2, ew3, eb3, cb, cn_ref, pt_ref,
               p_out, tok_out, loss_out):
    i = pl.program_id(0)
    t = praw_ref[...] / 255.0 * 2.0 - 1.0
    z = _gelu(jnp.dot(t, ew1[...], preferred_element_type=jnp.float32) + eb1[...])
    z = _gelu(jnp.dot(z, ew2[...], preferred_element_type=jnp.float32) + eb2[...])
    z_e = jnp.dot(z, ew3[...], preferred_element_type=jnp.float32) + eb3[...]

    score = jnp.dot(z_e, cb[...].T, preferred_element_type=jnp.float32)
    g = cn_ref[...] - 2.0 * score

    m = jnp.min(g, axis=-1, keepdims=True)
    iota = jax.lax.broadcasted_iota(jnp.int32, g.shape, 1)
    tok = jnp.min(jnp.where(g == m, iota, VOCAB), axis=-1)
    tok_out[0, 0, :] = tok

    onehot = (iota == tok[:, None]).astype(jnp.float32)
    p = jnp.dot(onehot, pt_ref[...], preferred_element_type=jnp.float32)
    p_out[...] = p

    zn = jnp.sum(z_e * z_e, axis=-1, keepdims=True)
    vq_sum = jnp.sum(zn + m)
    rec_sum = jnp.sum((p - t) ** 2)

    @pl.when(i == 0)
    def _init():
        loss_out[...] = jnp.zeros_like(loss_out)

    upd = jnp.concatenate([rec_sum.reshape(1, 1), vq_sum.reshape(1, 1)], axis=1)
    loss_out[...] += upd


def kernel(frames, enc_w1, enc_b1, enc_w2, enc_b2, enc_w3, enc_b3, codebook,
           dec_w1, dec_b1, dec_w2, dec_b2, dec_w3, dec_b3):
    praw = frames.astype(jnp.float32).reshape(B, Hp, PS, Wp, PS, C)
    praw = praw.transpose(0, 1, 3, 2, 4, 5).reshape(R, PD)

    full = lambda shape: pl.BlockSpec(shape, lambda i: (0,) * len(shape))

    ptable, cn = pl.pallas_call(
        _table_body,
        grid=(1,),
        in_specs=[full((VOCAB, D)), full((D, D)), full((1, D)), full((D, D)),
                  full((1, D)), full((D, PD)), full((1, PD))],
        out_specs=(full((VOCAB, PD)), full((1, VOCAB))),
        out_shape=(jax.ShapeDtypeStruct((VOCAB, PD), jnp.float32),
                   jax.ShapeDtypeStruct((1, VOCAB), jnp.float32)),
    )(codebook, dec_w1, dec_b1.reshape(1, D), dec_w2, dec_b2.reshape(1, D),
      dec_w3, dec_b3.reshape(1, PD))

    bspecs = [
        pl.BlockSpec((BLK, PD), lambda i: (i, 0)),
        full((PD, D)), full((1, D)),
        full((D, D)), full((1, D)),
        full((D, D)), full((1, D)),
        full((VOCAB, D)),
        full((1, VOCAB)),
        full((VOCAB, PD)),
    ]
    out_shapes = (
        jax.ShapeDtypeStruct((R, PD), jnp.float32),
        jax.ShapeDtypeStruct((G, 1, BLK), jnp.int32),
        jax.ShapeDtypeStruct((1, 2), jnp.float32),
    )
    out_specs = (
        pl.BlockSpec((BLK, PD), lambda i: (i, 0)),
        pl.BlockSpec((1, 1, BLK), lambda i: (i, 0, 0)),
        pl.BlockSpec((1, 2), lambda i: (0, 0)),
    )
    p_full, tok3, sums = pl.pallas_call(
        _main_body,
        grid=(G,),
        in_specs=bspecs,
        out_specs=out_specs,
        out_shape=out_shapes,
    )(praw, enc_w1, enc_b1.reshape(1, D), enc_w2, enc_b2.reshape(1, D),
      enc_w3, enc_b3.reshape(1, D), codebook, cn, ptable)

    tokens = tok3.reshape(B, N)
    recon = p_full.reshape(B, Hp, Wp, PS, PS, C).transpose(0, 1, 3, 2, 4, 5)
    recon = recon.reshape(B, H, W, C)
    recon_loss = sums[0, 0] / (B * H * W * C)
    vq_loss = sums[0, 1] / (R * D)
    return (recon, tokens, recon_loss, vq_loss, vq_loss)
